# baseline (speedup 1.0000x reference)
import functools

import jax
import jax.numpy as jnp
from jax import lax
from jax.experimental import pallas as pl
from jax.experimental.pallas import tpu as pltpu
from jax.experimental.pallas import tpu_sc as plsc

N_CODES = 8192
CODE_DIM = 256

_MT = 256
_NT = 8192

_I32_MAX = jnp.iinfo(jnp.int32).max


def _dist_argmin_body(cbnorm_ref, zs_ref, cb_ref, idx_ref,
                      minval_ref, minidx_ref):
    j = pl.program_id(0)
    i = pl.program_id(1)
    nj = pl.num_programs(0)
    row = pl.ds(i * _MT, _MT)

    zs = zs_ref[...]
    cb = cb_ref[...]
    m2 = lax.dot_general(zs, cb, (((1,), (1,)), ((), ())),
                         preferred_element_type=jnp.float32)
    d = cbnorm_ref[...] + m2

    loc_min = jnp.min(d, axis=1, keepdims=True)
    gidx = lax.broadcasted_iota(jnp.int32, d.shape, 1) + j * _NT
    loc_idx = jnp.min(jnp.where(d == loc_min, gidx, _I32_MAX),
                      axis=1, keepdims=True)

    @pl.when(j == 0)
    def _init():
        minval_ref[row, :] = loc_min
        minidx_ref[row, :] = loc_idx

    @pl.when(j > 0)
    def _update():
        prev = minval_ref[row, :]
        better = loc_min < prev
        minval_ref[row, :] = jnp.where(better, loc_min, prev)
        minidx_ref[row, :] = jnp.where(better, loc_idx, minidx_ref[row, :])

    @pl.when(j == nj - 1)
    def _emit():
        idx_ref[...] = minidx_ref[row, :]


def _argmin_indices(cbnorm, zs, codebook):
    m = zs.shape[0]
    grid = (N_CODES // _NT, m // _MT)
    return pl.pallas_call(
        _dist_argmin_body,
        grid=grid,
        in_specs=[
            pl.BlockSpec((1, _NT), lambda j, i: (0, j)),
            pl.BlockSpec((_MT, CODE_DIM), lambda j, i: (i, 0)),
            pl.BlockSpec((_NT, CODE_DIM), lambda j, i: (j, 0)),
        ],
        out_specs=pl.BlockSpec((_MT, 1), lambda j, i: (i, 0)),
        out_shape=jax.ShapeDtypeStruct((m, 1), jnp.int32),
        scratch_shapes=[
            pltpu.VMEM((m, 1), jnp.float32),
            pltpu.VMEM((m, 1), jnp.int32),
        ],
        compiler_params=pltpu.CompilerParams(
            dimension_semantics=("arbitrary", "parallel")),
    )(cbnorm, zs, codebook)


@functools.lru_cache(maxsize=None)
def _make_sc_gather(num_rows):
    info = plsc.get_sparse_core_info()
    nw = info.num_cores * info.num_subcores
    rows_per_w = num_rows // nw
    chunk = 128
    nchunks = rows_per_w // chunk
    mesh = plsc.VectorSubcoreMesh(core_axis_name="c", subcore_axis_name="s")

    @functools.partial(
        pl.kernel, mesh=mesh,
        out_type=jax.ShapeDtypeStruct((num_rows, CODE_DIM), jnp.float32),
        scratch_types=[
            pltpu.VMEM((chunk,), jnp.int32),
            pltpu.VMEM((chunk, CODE_DIM), jnp.float32),
            pltpu.SemaphoreType.DMA,
        ],
    )
    def gather(table_hbm, idx_hbm, out_hbm, idx_v, rows_v, sem):
        wid = lax.axis_index("s") * info.num_cores + lax.axis_index("c")
        base = wid * rows_per_w
        for c in range(nchunks):
            b = base + c * chunk
            pltpu.sync_copy(idx_hbm.at[pl.ds(b, chunk)], idx_v)
            pltpu.async_copy(table_hbm.at[idx_v], rows_v, sem).wait()
            pltpu.sync_copy(rows_v, out_hbm.at[pl.ds(b, chunk)])

    return gather


def kernel(z, codebook):
    b, t, dim = z.shape
    m = b * t
    cbnorm = jnp.sum(codebook ** 2, axis=-1).reshape(1, N_CODES)
    zs = (-2.0) * z.reshape(m, dim)

    idx = _argmin_indices(cbnorm, zs, codebook)
    idx_flat = idx.reshape(m)
    z_q = _make_sc_gather(m)(codebook, idx_flat)
    return (z_q.reshape(b, t, dim), idx_flat.reshape(b, t))


# speedup vs baseline: 1.0962x; 1.0962x over previous
import functools

import jax
import jax.numpy as jnp
from jax import lax
from jax.experimental import pallas as pl
from jax.experimental.pallas import tpu as pltpu
from jax.experimental.pallas import tpu_sc as plsc

N_CODES = 8192
CODE_DIM = 256

_MT = 1024
_NT = 8192

_I32_MAX = jnp.iinfo(jnp.int32).max


def _dist_argmin_body(cbnorm_ref, zs_ref, cb_ref, idx_ref,
                      minval_ref, minidx_ref):
    j = pl.program_id(0)
    i = pl.program_id(1)
    nj = pl.num_programs(0)
    row = pl.ds(i * _MT, _MT)

    zs = zs_ref[...]
    cb = cb_ref[...]
    m2 = lax.dot_general(zs, cb, (((1,), (1,)), ((), ())),
                         preferred_element_type=jnp.float32)
    d = cbnorm_ref[...] + m2

    loc_min = jnp.min(d, axis=1, keepdims=True)
    gidx = lax.broadcasted_iota(jnp.int32, d.shape, 1) + j * _NT
    loc_idx = jnp.min(jnp.where(d == loc_min, gidx, _I32_MAX),
                      axis=1, keepdims=True)

    @pl.when(j == 0)
    def _init():
        minval_ref[row, :] = loc_min
        minidx_ref[row, :] = loc_idx

    @pl.when(j > 0)
    def _update():
        prev = minval_ref[row, :]
        better = loc_min < prev
        minval_ref[row, :] = jnp.where(better, loc_min, prev)
        minidx_ref[row, :] = jnp.where(better, loc_idx, minidx_ref[row, :])

    @pl.when(j == nj - 1)
    def _emit():
        idx_ref[...] = minidx_ref[row, :]


def _argmin_indices(cbnorm, zs, codebook):
    m = zs.shape[0]
    grid = (N_CODES // _NT, m // _MT)
    return pl.pallas_call(
        _dist_argmin_body,
        grid=grid,
        in_specs=[
            pl.BlockSpec((1, _NT), lambda j, i: (0, j)),
            pl.BlockSpec((_MT, CODE_DIM), lambda j, i: (i, 0)),
            pl.BlockSpec((_NT, CODE_DIM), lambda j, i: (j, 0)),
        ],
        out_specs=pl.BlockSpec((_MT, 1), lambda j, i: (i, 0)),
        out_shape=jax.ShapeDtypeStruct((m, 1), jnp.int32),
        scratch_shapes=[
            pltpu.VMEM((m, 1), jnp.float32),
            pltpu.VMEM((m, 1), jnp.int32),
        ],
        compiler_params=pltpu.CompilerParams(
            dimension_semantics=("arbitrary", "parallel")),
    )(cbnorm, zs, codebook)


@functools.lru_cache(maxsize=None)
def _make_sc_gather(num_rows):
    info = plsc.get_sparse_core_info()
    nw = info.num_cores * info.num_subcores
    rows_per_w = num_rows // nw
    chunk = 128
    nchunks = rows_per_w // chunk
    mesh = plsc.VectorSubcoreMesh(core_axis_name="c", subcore_axis_name="s")

    @functools.partial(
        pl.kernel, mesh=mesh,
        out_type=jax.ShapeDtypeStruct((num_rows, CODE_DIM), jnp.float32),
        scratch_types=[
            pltpu.VMEM((chunk,), jnp.int32),
            pltpu.VMEM((chunk, CODE_DIM), jnp.float32),
            pltpu.SemaphoreType.DMA,
        ],
    )
    def gather(table_hbm, idx_hbm, out_hbm, idx_v, rows_v, sem):
        wid = lax.axis_index("s") * info.num_cores + lax.axis_index("c")
        base = wid * rows_per_w
        for c in range(nchunks):
            b = base + c * chunk
            pltpu.sync_copy(idx_hbm.at[pl.ds(b, chunk)], idx_v)
            pltpu.async_copy(table_hbm.at[idx_v], rows_v, sem).wait()
            pltpu.sync_copy(rows_v, out_hbm.at[pl.ds(b, chunk)])

    return gather


def kernel(z, codebook):
    b, t, dim = z.shape
    m = b * t
    cbnorm = jnp.sum(codebook ** 2, axis=-1).reshape(1, N_CODES)
    zs = (-2.0) * z.reshape(m, dim)

    idx = _argmin_indices(cbnorm, zs, codebook)
    idx_flat = idx.reshape(m)
    z_q = _make_sc_gather(m)(codebook, idx_flat)
    return (z_q.reshape(b, t, dim), idx_flat.reshape(b, t))
